# baseline (speedup 1.0000x reference)
import jax
import jax.numpy as jnp
from jax import lax
from jax.experimental import pallas as pl
from jax.experimental.pallas import tpu as pltpu
from jax.experimental.pallas import tpu_sc as plsc

_L = 16
_K = 128
_NCH = 8
_B = 64
_BPT = 2
_HW = 128 * 128


def _body(hps_flat, ind_hbm, mask_hbm, part_hbm,
          idx_v, gidx_v, vals_v, mask_v, part_v,
          sem_m, sem0, sem1):
    cid = lax.axis_index("c")
    sid = lax.axis_index("s")
    wid = cid * 16 + sid
    gsems = (sem0, sem1)

    c_ind = pltpu.make_async_copy(
        ind_hbm.at[pl.ds(wid * (_BPT * _K), _BPT * _K)], idx_v, sem_m)
    c_msk = pltpu.make_async_copy(
        mask_hbm.at[pl.ds(wid * _BPT, _BPT)], mask_v, sem_m)
    c_ind.start()
    c_msk.start()
    c_ind.wait()

    gathers = [[] for _ in range(_BPT)]
    for bb in range(_BPT):
        base = (wid * _BPT + bb) * (16 * _HW)
        for t in range(_K // _L):
            p = idx_v[pl.ds(bb * _K + t * _L, _L)] + base
            for ci in range(_NCH):
                gidx_v[bb * _NCH + ci, pl.ds(t * _L, _L)] = \
                    p + ci * (2 * _HW)
        for ci in range(_NCH):
            r = bb * _NCH + ci
            c = pltpu.make_async_copy(
                hps_flat.at[gidx_v.at[r]],
                vals_v.at[pl.ds(r * _K, _K)], gsems[bb])
            c.start()
            gathers[bb].append(c)
    c_msk.wait()

    acc = jnp.zeros((_L,), jnp.float32)
    msum = jnp.zeros((_L,), jnp.float32)
    for bb in range(_BPT):
        for c in gathers[bb]:
            c.wait()
        for j in range(4):
            for t in range(_K // _L):
                xb = vals_v[pl.ds((bb * _NCH + j) * _K + t * _L, _L)]
                xu = vals_v[pl.ds((bb * _NCH + j + 4) * _K + t * _L, _L)]
                mb = mask_v[bb, j, pl.ds(t * _L, _L)]
                mu = mask_v[bb, j + 4, pl.ds(t * _L, _L)]
                m = (mb * mu).astype(jnp.float32)
                acc = acc + jnp.abs(xu - xb) * m
                msum = msum + m

    part_v[0] = acc
    part_v[1] = msum
    pltpu.sync_copy(part_v, part_hbm.at[wid])


def _tc_combine(x_ref, o_ref):
    x = x_ref[...]
    i = lax.broadcasted_iota(jnp.int32, (1024,), 0)
    is_acc = (i & 16) == 0
    s = jnp.sum(jnp.where(is_acc, x, 0.0))
    m = jnp.sum(jnp.where(is_acc, 0.0, x))
    o_ref[0, 0] = s / (m + 0.0001)


@jax.jit
def _kp_align(hps_flat, ind_flat, mask_t):
    mesh = plsc.VectorSubcoreMesh(
        core_axis_name="c", subcore_axis_name="s", num_cores=2)
    fn = pl.kernel(
        _body,
        out_type=jax.ShapeDtypeStruct((32, 2, _L), jnp.float32),
        mesh=mesh,
        scratch_types=[
            pltpu.VMEM((_BPT * _K,), jnp.int32),
            pltpu.VMEM((_BPT * _NCH, _K), jnp.int32),
            pltpu.VMEM((_BPT * _NCH * _K,), jnp.float32),
            pltpu.VMEM((_BPT, _NCH, _K), jnp.int32),
            pltpu.VMEM((2, _L), jnp.float32),
            pltpu.SemaphoreType.DMA,
            pltpu.SemaphoreType.DMA,
            pltpu.SemaphoreType.DMA,
        ],
    )
    parts = fn(hps_flat, ind_flat, mask_t)
    loss = pl.pallas_call(
        _tc_combine,
        out_shape=jax.ShapeDtypeStruct((1, 1), jnp.float32),
        out_specs=pl.BlockSpec(memory_space=pltpu.SMEM),
    )(parts.reshape(32 * 2 * _L))
    return loss


def kernel(hps, ind, inv_mask):
    B, C, H, W = hps.shape
    hps_flat = hps.reshape(B * C * H * W)
    mask_t = jnp.transpose(inv_mask[:, :, 0:16:2], (0, 2, 1))
    loss = _kp_align(hps_flat, ind.reshape(B * _K), mask_t)
    return loss[0, 0]


# speedup vs baseline: 1.0478x; 1.0478x over previous
import jax
import jax.numpy as jnp
from jax import lax
from jax.experimental import pallas as pl
from jax.experimental.pallas import tpu as pltpu
from jax.experimental.pallas import tpu_sc as plsc

_L = 16
_K = 128
_NCH = 8
_B = 64
_BPT = 2
_HW = 128 * 128


def _body(hps_flat, ind_hbm, mask_hbm, part_hbm,
          idx_v, gidx_v, vals_v, mask_v, part_v,
          sem_m, sem0, sem1):
    cid = lax.axis_index("c")
    sid = lax.axis_index("s")
    wid = cid * 16 + sid
    gsems = (sem0, sem1)

    c_ind = pltpu.make_async_copy(
        ind_hbm.at[pl.ds(wid * (_BPT * _K), _BPT * _K)], idx_v, sem_m)
    c_msk = pltpu.make_async_copy(
        mask_hbm.at[pl.ds(wid * _BPT, _BPT)], mask_v, sem_m)
    c_ind.start()
    c_msk.start()
    c_ind.wait()

    gathers = [[] for _ in range(_BPT)]
    for bb in range(_BPT):
        base = (wid * _BPT + bb) * (16 * _HW)
        for t in range(_K // _L):
            p = idx_v[pl.ds(bb * _K + t * _L, _L)] + base
            for ci in range(_NCH):
                gidx_v[bb * _NCH + ci, pl.ds(t * _L, _L)] = \
                    p + ci * (2 * _HW)
        for ci in range(_NCH):
            r = bb * _NCH + ci
            c = pltpu.make_async_copy(
                hps_flat.at[gidx_v.at[r]],
                vals_v.at[pl.ds(r * _K, _K)], gsems[bb])
            c.start()
            gathers[bb].append(c)
    c_msk.wait()

    acc = jnp.zeros((_L,), jnp.float32)
    msum = jnp.zeros((_L,), jnp.float32)
    for bb in range(_BPT):
        for c in gathers[bb]:
            c.wait()
        for j in range(4):
            for t in range(_K // _L):
                xb = vals_v[pl.ds((bb * _NCH + j) * _K + t * _L, _L)]
                xu = vals_v[pl.ds((bb * _NCH + j + 4) * _K + t * _L, _L)]
                mb = mask_v[bb, j, pl.ds(t * _L, _L)]
                mu = mask_v[bb, j + 4, pl.ds(t * _L, _L)]
                m = (mb * mu).astype(jnp.float32)
                acc = acc + jnp.abs(xu - xb) * m
                msum = msum + m

    part_v[0] = acc
    part_v[1] = msum
    w0 = pltpu.make_async_copy(
        part_v.at[0], part_hbm.at[wid, 0, pl.ds(0, _L)], sem_m)
    w1 = pltpu.make_async_copy(
        part_v.at[1], part_hbm.at[wid, 1, pl.ds(0, _L)], sem_m)
    w0.start()
    w1.start()
    w0.wait()
    w1.wait()


def _tc_combine(x_ref, o_ref):
    x = x_ref[...]
    lane = lax.broadcasted_iota(jnp.int32, (32, 2, 128), 2)
    x = jnp.where(lane < _L, x, 0.0)
    s = jnp.sum(x[:, 0, :])
    m = jnp.sum(x[:, 1, :])
    o_ref[0, 0] = s / (m + 0.0001)


@jax.jit
def _kp_align(hps_flat, ind_flat, mask_t):
    mesh = plsc.VectorSubcoreMesh(
        core_axis_name="c", subcore_axis_name="s", num_cores=2)
    fn = pl.kernel(
        _body,
        out_type=jax.ShapeDtypeStruct((32, 2, 128), jnp.float32),
        mesh=mesh,
        scratch_types=[
            pltpu.VMEM((_BPT * _K,), jnp.int32),
            pltpu.VMEM((_BPT * _NCH, _K), jnp.int32),
            pltpu.VMEM((_BPT * _NCH * _K,), jnp.float32),
            pltpu.VMEM((_BPT, _NCH, _K), jnp.int32),
            pltpu.VMEM((2, _L), jnp.float32),
            pltpu.SemaphoreType.DMA,
            pltpu.SemaphoreType.DMA,
            pltpu.SemaphoreType.DMA,
        ],
    )
    parts = fn(hps_flat, ind_flat, mask_t)
    loss = pl.pallas_call(
        _tc_combine,
        out_shape=jax.ShapeDtypeStruct((1, 1), jnp.float32),
        out_specs=pl.BlockSpec(memory_space=pltpu.SMEM),
    )(parts)
    return loss


def kernel(hps, ind, inv_mask):
    B, C, H, W = hps.shape
    hps_flat = hps.reshape(B * C * H * W)
    mask_t = jnp.transpose(inv_mask[:, :, 0:16:2], (0, 2, 1))
    loss = _kp_align(hps_flat, ind.reshape(B * _K), mask_t)
    return loss[0, 0]
